# Initial kernel scaffold; baseline (speedup 1.0000x reference)
import jax
import jax.numpy as jnp
from jax.experimental import pallas as pl


def kernel(x, seg, word_table, pos_table, seg_table, gamma, beta):
    raise NotImplementedError("write your pallas kernel here")


# speedup vs baseline: 1.9633x; 1.9633x over previous
import functools

import jax
import jax.numpy as jnp
from jax import lax
from jax.experimental import pallas as pl
from jax.experimental.pallas import tpu as pltpu
from jax.experimental.pallas import tpu_sc as plsc

_B, _S, _D = 4, 2048, 768
_PAD = 1
_EPS = 1e-12
_NC, _NS = 2, 16
_NW = _NC * _NS
_TOK = _B * _S
_TPW = _TOK // _NW
_K = 16
_NCH = _TPW // _K
_NBUF = 4
_TB = 256

_mesh = plsc.VectorSubcoreMesh(core_axis_name="c", subcore_axis_name="s")


@functools.partial(
    pl.kernel,
    out_type=jax.ShapeDtypeStruct((_TOK, _D), jnp.float32),
    mesh=_mesh,
    scratch_types=[
        pltpu.VMEM((_TPW,), jnp.int32),
        pltpu.VMEM((_NBUF, _K, _D), jnp.float32),
        pltpu.SemaphoreType.DMA,
        pltpu.SemaphoreType.DMA,
        pltpu.SemaphoreType.DMA,
        pltpu.SemaphoreType.DMA,
        pltpu.SemaphoreType.DMA,
        pltpu.SemaphoreType.DMA,
        pltpu.SemaphoreType.DMA,
        pltpu.SemaphoreType.DMA,
    ],
)
def _gather_rows(x_hbm, wt_hbm, out_hbm, idx_v, rows_v,
                 g0, g1, g2, g3, o0, o1, o2, o3):
    wid = lax.axis_index("s") * _NC + lax.axis_index("c")
    base = wid * _TPW
    gsems = (g0, g1, g2, g3)
    osems = (o0, o1, o2, o3)

    pltpu.sync_copy(x_hbm.at[pl.ds(base, _TPW)], idx_v)

    def gather_desc(buf, tk0):
        return pltpu.make_async_copy(
            wt_hbm.at[idx_v.at[pl.ds(tk0, _K)]], rows_v.at[buf], gsems[buf])

    def out_desc(buf, tk0):
        return pltpu.make_async_copy(
            rows_v.at[buf], out_hbm.at[pl.ds(base + tk0, _K)], osems[buf])

    for b in range(_NBUF - 1):
        gather_desc(b, b * _K).start()

    def step(c4, carry0):
        for sub in range(_NBUF):
            c = c4 * _NBUF + sub
            tk0 = pl.multiple_of(c * _K, _K)
            gather_desc(sub, tk0).wait()
            out_desc(sub, tk0).start()

            @pl.when(c + _NBUF - 1 < _NCH)
            def _():
                @pl.when(c >= 1)
                def _():
                    prev = pl.multiple_of((c - 1) * _K, _K)
                    out_desc((sub + _NBUF - 1) % _NBUF, prev).wait()
                nxt = pl.multiple_of((c + _NBUF - 1) * _K, _K)
                gather_desc((sub + _NBUF - 1) % _NBUF, nxt).start()
        return carry0

    lax.fori_loop(0, _NCH // _NBUF, step, 0)

    for c in range(_NCH - _NBUF, _NCH):
        out_desc(c % _NBUF, c * _K).wait()


def _ln_body(rows_ref, pos_ref, segid_ref, segtab_ref, g_ref, b_ref, o_ref):
    h = rows_ref[...] + pos_ref[...]
    sv = segid_ref[...]
    s0 = segtab_ref[0:1, :]
    s1 = segtab_ref[1:2, :]
    h = h + jnp.where(sv > 0, s1, s0)
    mean = jnp.mean(h, axis=1, keepdims=True)
    var = jnp.mean(jnp.square(h - mean), axis=1, keepdims=True)
    hn = (h - mean) * lax.rsqrt(var + _EPS)
    o_ref[...] = hn * g_ref[...] + b_ref[...]


_ln_call = pl.pallas_call(
    _ln_body,
    out_shape=jax.ShapeDtypeStruct((_TOK, _D), jnp.float32),
    grid=(_TOK // _TB,),
    in_specs=[
        pl.BlockSpec((_TB, _D), lambda i: (i, 0)),
        pl.BlockSpec((_TB, _D), lambda i: (i % (_S // _TB), 0)),
        pl.BlockSpec((_TB, 1), lambda i: (i, 0)),
        pl.BlockSpec((2, _D), lambda i: (0, 0)),
        pl.BlockSpec((1, _D), lambda i: (0, 0)),
        pl.BlockSpec((1, _D), lambda i: (0, 0)),
    ],
    out_specs=pl.BlockSpec((_TB, _D), lambda i: (i, 0)),
)


def kernel(x, seg, word_table, pos_table, seg_table, gamma, beta):
    xf = x.reshape(-1).astype(jnp.int32)
    rows = _gather_rows(xf, word_table)
    pos_used = lax.slice_in_dim(pos_table, _PAD + 1, _PAD + 1 + _S, axis=0)
    out = _ln_call(rows, pos_used, seg.reshape(-1, 1).astype(jnp.int32),
                   seg_table, gamma.reshape(1, _D), beta.reshape(1, _D))
    return out.reshape(_B, _S, _D)


# speedup vs baseline: 2.0523x; 1.0453x over previous
import functools

import jax
import jax.numpy as jnp
from jax import lax
from jax.experimental import pallas as pl
from jax.experimental.pallas import tpu as pltpu
from jax.experimental.pallas import tpu_sc as plsc

_B, _S, _D = 4, 2048, 768
_PAD = 1
_EPS = 1e-12
_NC, _NS = 2, 16
_NW = _NC * _NS
_TOK = _B * _S
_TPW = _TOK // _NW
_K = 16
_NCH = _TPW // _K
_NBUF = 4
_TB = 256

_mesh = plsc.VectorSubcoreMesh(core_axis_name="c", subcore_axis_name="s")


@functools.partial(
    pl.kernel,
    out_type=jax.ShapeDtypeStruct((_TOK, _D), jnp.float32),
    mesh=_mesh,
    scratch_types=[
        pltpu.VMEM((_TPW,), jnp.int32),
        pltpu.VMEM((_NBUF, _K, _D), jnp.float32),
        pltpu.SemaphoreType.DMA,
        pltpu.SemaphoreType.DMA,
        pltpu.SemaphoreType.DMA,
        pltpu.SemaphoreType.DMA,
        pltpu.SemaphoreType.DMA,
        pltpu.SemaphoreType.DMA,
        pltpu.SemaphoreType.DMA,
        pltpu.SemaphoreType.DMA,
    ],
)
def _gather_rows(x_hbm, wt_hbm, out_hbm, idx_v, rows_v,
                 g0, g1, g2, g3, o0, o1, o2, o3):
    wid = lax.axis_index("s") * _NC + lax.axis_index("c")
    base = wid * _TPW
    gsems = (g0, g1, g2, g3)
    osems = (o0, o1, o2, o3)

    pltpu.sync_copy(x_hbm.at[pl.ds(base, _TPW)], idx_v)

    def gather_desc(buf, tk0):
        return pltpu.make_async_copy(
            wt_hbm.at[idx_v.at[pl.ds(tk0, _K)]], rows_v.at[buf], gsems[buf])

    def out_desc(buf, tk0):
        return pltpu.make_async_copy(
            rows_v.at[buf], out_hbm.at[pl.ds(base + tk0, _K)], osems[buf])

    for b in range(_NBUF - 1):
        gather_desc(b, b * _K).start()

    def step(c4, carry0):
        for sub in range(_NBUF):
            c = c4 * _NBUF + sub
            tk0 = pl.multiple_of(c * _K, _K)
            gather_desc(sub, tk0).wait()
            out_desc(sub, tk0).start()

            @pl.when(c + _NBUF - 1 < _NCH)
            def _():
                @pl.when(c >= 1)
                def _():
                    prev = pl.multiple_of((c - 1) * _K, _K)
                    out_desc((sub + _NBUF - 1) % _NBUF, prev).wait()
                nxt = pl.multiple_of((c + _NBUF - 1) * _K, _K)
                gather_desc((sub + _NBUF - 1) % _NBUF, nxt).start()
        return carry0

    lax.fori_loop(0, _NCH // _NBUF, step, 0)

    for c in range(_NCH - _NBUF, _NCH):
        out_desc(c % _NBUF, c * _K).wait()


_PB = _S // _TB


def _ln_body(rows_ref, pos_ref, posn_ref, segid_ref, segtab_ref, g_ref,
             b_ref, o_ref):
    pos = jnp.concatenate(
        [pos_ref[_PAD + 1:, :], posn_ref[:_PAD + 1, :]], axis=0)
    h = rows_ref[...] + pos
    sv = segid_ref[...]
    s0 = segtab_ref[0:1, :]
    s1 = segtab_ref[1:2, :]
    h = h + jnp.where(sv > 0, s1, s0)
    mean = jnp.mean(h, axis=1, keepdims=True)
    var = jnp.mean(h * h, axis=1, keepdims=True) - mean * mean
    hn = (h - mean) * lax.rsqrt(var + _EPS)
    o_ref[...] = hn * g_ref[...] + b_ref[...]


_ln_call = pl.pallas_call(
    _ln_body,
    out_shape=jax.ShapeDtypeStruct((_TOK, _D), jnp.float32),
    grid=(_TOK // _TB,),
    in_specs=[
        pl.BlockSpec((_TB, _D), lambda i: (i, 0)),
        pl.BlockSpec((_TB, _D), lambda i: (i % _PB, 0)),
        pl.BlockSpec((8, _D),
                     lambda i: ((i % _PB) * (_TB // 8) + _TB // 8, 0)),
        pl.BlockSpec((_TB, 1), lambda i: (i, 0)),
        pl.BlockSpec((2, _D), lambda i: (0, 0)),
        pl.BlockSpec((1, _D), lambda i: (0, 0)),
        pl.BlockSpec((1, _D), lambda i: (0, 0)),
    ],
    out_specs=pl.BlockSpec((_TB, _D), lambda i: (i, 0)),
)


def kernel(x, seg, word_table, pos_table, seg_table, gamma, beta):
    xf = x.reshape(-1).astype(jnp.int32)
    rows = _gather_rows(xf, word_table)
    out = _ln_call(rows, pos_table, pos_table,
                   seg.reshape(-1, 1).astype(jnp.int32),
                   seg_table, gamma.reshape(1, _D), beta.reshape(1, _D))
    return out.reshape(_B, _S, _D)


# speedup vs baseline: 2.0813x; 1.0141x over previous
import functools

import jax
import jax.numpy as jnp
from jax import lax
from jax.experimental import pallas as pl
from jax.experimental.pallas import tpu as pltpu
from jax.experimental.pallas import tpu_sc as plsc

_B, _S, _D = 4, 2048, 768
_PAD = 1
_EPS = 1e-12
_NC, _NS = 2, 16
_NW = _NC * _NS
_TOK = _B * _S
_TPW = _TOK // _NW
_K = 16
_NCH = _TPW // _K
_NBUF = 4
_TB = 256

_mesh = plsc.VectorSubcoreMesh(core_axis_name="c", subcore_axis_name="s")


@functools.partial(
    pl.kernel,
    out_type=jax.ShapeDtypeStruct((_TOK, _D), jnp.float32),
    mesh=_mesh,
    scratch_types=[
        pltpu.VMEM((_TPW,), jnp.int32),
        pltpu.VMEM((_NBUF, _K, _D), jnp.float32),
        pltpu.SemaphoreType.DMA,
        pltpu.SemaphoreType.DMA,
        pltpu.SemaphoreType.DMA,
        pltpu.SemaphoreType.DMA,
        pltpu.SemaphoreType.DMA,
        pltpu.SemaphoreType.DMA,
        pltpu.SemaphoreType.DMA,
        pltpu.SemaphoreType.DMA,
    ],
)
def _gather_rows(x_hbm, wt_hbm, out_hbm, idx_v, rows_v,
                 g0, g1, g2, g3, o0, o1, o2, o3):
    wid = lax.axis_index("s") * _NC + lax.axis_index("c")
    base = wid * _TPW
    gsems = (g0, g1, g2, g3)
    osems = (o0, o1, o2, o3)

    pltpu.sync_copy(x_hbm.at[pl.ds(base, _TPW)], idx_v)

    def gather_desc(buf, tk0):
        return pltpu.make_async_copy(
            wt_hbm.at[idx_v.at[pl.ds(tk0, _K)]], rows_v.at[buf], gsems[buf])

    def out_desc(buf, tk0):
        return pltpu.make_async_copy(
            rows_v.at[buf], out_hbm.at[pl.ds(base + tk0, _K)], osems[buf])

    for b in range(_NBUF - 1):
        gather_desc(b, b * _K).start()

    def step(c4, carry0):
        for sub in range(_NBUF):
            c = c4 * _NBUF + sub
            tk0 = pl.multiple_of(c * _K, _K)
            gather_desc(sub, tk0).wait()
            out_desc(sub, tk0).start()

            @pl.when(c + _NBUF - 1 < _NCH)
            def _():
                @pl.when(c >= 1)
                def _():
                    prev = pl.multiple_of((c - 1) * _K, _K)
                    out_desc((sub + _NBUF - 1) % _NBUF, prev).wait()
                nxt = pl.multiple_of((c + _NBUF - 1) * _K, _K)
                gather_desc((sub + _NBUF - 1) % _NBUF, nxt).start()
        return carry0

    lax.fori_loop(0, _NCH // _NBUF, step, 0)

    for c in range(_NCH - _NBUF, _NCH):
        out_desc(c % _NBUF, c * _K).wait()


_PB = _S // _TB


def _ln_body(rows_ref, pos_ref, posn_ref, segid_ref, segtab_ref, g_ref,
             b_ref, o_ref):
    pos = jnp.concatenate(
        [pos_ref[_PAD + 1:, :], posn_ref[:_PAD + 1, :]], axis=0)
    h = rows_ref[...] + pos
    sv = segid_ref[...]
    s0 = segtab_ref[0:1, :]
    s1 = segtab_ref[1:2, :]
    h = h + jnp.where(sv > 0, s1, s0)
    mean = jnp.mean(h, axis=1, keepdims=True)
    var = jnp.mean(h * h, axis=1, keepdims=True) - mean * mean
    hn = (h - mean) * lax.rsqrt(var + _EPS)
    o_ref[...] = hn * g_ref[...] + b_ref[...]


_ln_call = pl.pallas_call(
    _ln_body,
    out_shape=jax.ShapeDtypeStruct((_TOK, _D), jnp.float32),
    grid=(_PB, _B),
    in_specs=[
        pl.BlockSpec((_TB, _D), lambda p, b: (b * _PB + p, 0)),
        pl.BlockSpec((_TB, _D), lambda p, b: (p, 0)),
        pl.BlockSpec((8, _D),
                     lambda p, b: (p * (_TB // 8) + _TB // 8, 0)),
        pl.BlockSpec((_TB, 1), lambda p, b: (b * _PB + p, 0)),
        pl.BlockSpec((2, _D), lambda p, b: (0, 0)),
        pl.BlockSpec((1, _D), lambda p, b: (0, 0)),
        pl.BlockSpec((1, _D), lambda p, b: (0, 0)),
    ],
    out_specs=pl.BlockSpec((_TB, _D), lambda p, b: (b * _PB + p, 0)),
)


def kernel(x, seg, word_table, pos_table, seg_table, gamma, beta):
    xf = x.reshape(-1).astype(jnp.int32)
    rows = _gather_rows(xf, word_table)
    out = _ln_call(rows, pos_table, pos_table,
                   seg.reshape(-1, 1).astype(jnp.int32),
                   seg_table, gamma.reshape(1, _D), beta.reshape(1, _D))
    return out.reshape(_B, _S, _D)


# speedup vs baseline: 2.3848x; 1.1458x over previous
import functools

import jax
import jax.numpy as jnp
from jax import lax
from jax.experimental import pallas as pl
from jax.experimental.pallas import tpu as pltpu
from jax.experimental.pallas import tpu_sc as plsc

_B, _S, _D = 4, 2048, 768
_PAD = 1
_EPS = 1e-12
_NC, _NS = 2, 16
_NW = _NC * _NS
_TOK = _B * _S
_TPW = _TOK // _NW
_K = 16
_NCH = _TPW // _K
_NBUF = 4
_TB = 512

_mesh = plsc.VectorSubcoreMesh(core_axis_name="c", subcore_axis_name="s")


@functools.partial(
    pl.kernel,
    out_type=jax.ShapeDtypeStruct((_TOK, _D), jnp.float32),
    mesh=_mesh,
    scratch_types=[
        pltpu.VMEM((_TPW,), jnp.int32),
        pltpu.VMEM((_NBUF, _K, _D), jnp.float32),
        pltpu.SemaphoreType.DMA,
        pltpu.SemaphoreType.DMA,
        pltpu.SemaphoreType.DMA,
        pltpu.SemaphoreType.DMA,
        pltpu.SemaphoreType.DMA,
        pltpu.SemaphoreType.DMA,
        pltpu.SemaphoreType.DMA,
        pltpu.SemaphoreType.DMA,
    ],
)
def _gather_rows(x_hbm, wt_hbm, out_hbm, idx_v, rows_v,
                 g0, g1, g2, g3, o0, o1, o2, o3):
    wid = lax.axis_index("s") * _NC + lax.axis_index("c")
    base = wid * _TPW
    gsems = (g0, g1, g2, g3)
    osems = (o0, o1, o2, o3)

    pltpu.sync_copy(x_hbm.at[pl.ds(base, _TPW)], idx_v)

    def gather_desc(buf, tk0):
        return pltpu.make_async_copy(
            wt_hbm.at[idx_v.at[pl.ds(tk0, _K)]], rows_v.at[buf], gsems[buf])

    def out_desc(buf, tk0):
        return pltpu.make_async_copy(
            rows_v.at[buf], out_hbm.at[pl.ds(base + tk0, _K)], osems[buf])

    for b in range(_NBUF - 1):
        gather_desc(b, b * _K).start()

    def step(c4, carry0):
        for sub in range(_NBUF):
            c = c4 * _NBUF + sub
            tk0 = pl.multiple_of(c * _K, _K)
            gather_desc(sub, tk0).wait()
            out_desc(sub, tk0).start()

            @pl.when(c + _NBUF - 1 < _NCH)
            def _():
                @pl.when(c >= 1)
                def _():
                    prev = pl.multiple_of((c - 1) * _K, _K)
                    out_desc((sub + _NBUF - 1) % _NBUF, prev).wait()
                nxt = pl.multiple_of((c + _NBUF - 1) * _K, _K)
                gather_desc((sub + _NBUF - 1) % _NBUF, nxt).start()
        return carry0

    lax.fori_loop(0, _NCH // _NBUF, step, 0)

    for c in range(_NCH - _NBUF, _NCH):
        out_desc(c % _NBUF, c * _K).wait()


_PB = _S // _TB


def _ln_body(rows_ref, pos_ref, posn_ref, segid_ref, segtab_ref, g_ref,
             b_ref, o_ref):
    pos = jnp.concatenate(
        [pos_ref[_PAD + 1:, :], posn_ref[:_PAD + 1, :]], axis=0)
    h = rows_ref[...] + pos
    sv = segid_ref[...]
    s0 = segtab_ref[0:1, :]
    s1 = segtab_ref[1:2, :]
    h = h + jnp.where(sv > 0, s1, s0)
    mean = jnp.mean(h, axis=1, keepdims=True)
    var = jnp.mean(h * h, axis=1, keepdims=True) - mean * mean
    hn = (h - mean) * lax.rsqrt(var + _EPS)
    o_ref[...] = hn * g_ref[...] + b_ref[...]


_ln_call = pl.pallas_call(
    _ln_body,
    out_shape=jax.ShapeDtypeStruct((_TOK, _D), jnp.float32),
    grid=(_PB, _B),
    in_specs=[
        pl.BlockSpec((_TB, _D), lambda p, b: (b * _PB + p, 0)),
        pl.BlockSpec((_TB, _D), lambda p, b: (p, 0)),
        pl.BlockSpec((8, _D),
                     lambda p, b: (p * (_TB // 8) + _TB // 8, 0)),
        pl.BlockSpec((_TB, 1), lambda p, b: (b * _PB + p, 0)),
        pl.BlockSpec((2, _D), lambda p, b: (0, 0)),
        pl.BlockSpec((1, _D), lambda p, b: (0, 0)),
        pl.BlockSpec((1, _D), lambda p, b: (0, 0)),
    ],
    out_specs=pl.BlockSpec((_TB, _D), lambda p, b: (b * _PB + p, 0)),
)


def kernel(x, seg, word_table, pos_table, seg_table, gamma, beta):
    xf = x.reshape(-1).astype(jnp.int32)
    rows = _gather_rows(xf, word_table)
    out = _ln_call(rows, pos_table, pos_table,
                   seg.reshape(-1, 1).astype(jnp.int32),
                   seg_table, gamma.reshape(1, _D), beta.reshape(1, _D))
    return out.reshape(_B, _S, _D)


# speedup vs baseline: 2.5065x; 1.0510x over previous
import functools

import jax
import jax.numpy as jnp
from jax import lax
from jax.experimental import pallas as pl
from jax.experimental.pallas import tpu as pltpu
from jax.experimental.pallas import tpu_sc as plsc

_B, _S, _D = 4, 2048, 768
_PAD = 1
_EPS = 1e-12
_NC, _NS = 2, 16
_NW = _NC * _NS
_TOK = _B * _S
_TPW = _TOK // _NW
_K = 16
_NCH = _TPW // _K
_NBUF = 4
_TB = 1024

_mesh = plsc.VectorSubcoreMesh(core_axis_name="c", subcore_axis_name="s")


@functools.partial(
    pl.kernel,
    out_type=jax.ShapeDtypeStruct((_TOK, _D), jnp.float32),
    mesh=_mesh,
    scratch_types=[
        pltpu.VMEM((_TPW,), jnp.int32),
        pltpu.VMEM((_NBUF, _K, _D), jnp.float32),
        pltpu.SemaphoreType.DMA,
        pltpu.SemaphoreType.DMA,
        pltpu.SemaphoreType.DMA,
        pltpu.SemaphoreType.DMA,
        pltpu.SemaphoreType.DMA,
        pltpu.SemaphoreType.DMA,
        pltpu.SemaphoreType.DMA,
        pltpu.SemaphoreType.DMA,
    ],
)
def _gather_rows(x_hbm, wt_hbm, out_hbm, idx_v, rows_v,
                 g0, g1, g2, g3, o0, o1, o2, o3):
    wid = lax.axis_index("s") * _NC + lax.axis_index("c")
    base = wid * _TPW
    gsems = (g0, g1, g2, g3)
    osems = (o0, o1, o2, o3)

    pltpu.sync_copy(x_hbm.at[pl.ds(base, _TPW)], idx_v)

    def gather_desc(buf, tk0):
        return pltpu.make_async_copy(
            wt_hbm.at[idx_v.at[pl.ds(tk0, _K)]], rows_v.at[buf], gsems[buf])

    def out_desc(buf, tk0):
        return pltpu.make_async_copy(
            rows_v.at[buf], out_hbm.at[pl.ds(base + tk0, _K)], osems[buf])

    for b in range(_NBUF - 1):
        gather_desc(b, b * _K).start()

    def step(c4, carry0):
        for sub in range(_NBUF):
            c = c4 * _NBUF + sub
            tk0 = pl.multiple_of(c * _K, _K)
            gather_desc(sub, tk0).wait()
            out_desc(sub, tk0).start()

            @pl.when(c + _NBUF - 1 < _NCH)
            def _():
                @pl.when(c >= 1)
                def _():
                    prev = pl.multiple_of((c - 1) * _K, _K)
                    out_desc((sub + _NBUF - 1) % _NBUF, prev).wait()
                nxt = pl.multiple_of((c + _NBUF - 1) * _K, _K)
                gather_desc((sub + _NBUF - 1) % _NBUF, nxt).start()
        return carry0

    lax.fori_loop(0, _NCH // _NBUF, step, 0)

    for c in range(_NCH - _NBUF, _NCH):
        out_desc(c % _NBUF, c * _K).wait()


_PB = _S // _TB


def _ln_body(rows_ref, pos_ref, posn_ref, segid_ref, segtab_ref, g_ref,
             b_ref, o_ref):
    pos = jnp.concatenate(
        [pos_ref[_PAD + 1:, :], posn_ref[:_PAD + 1, :]], axis=0)
    h = rows_ref[...] + pos
    sv = segid_ref[...]
    s0 = segtab_ref[0:1, :]
    s1 = segtab_ref[1:2, :]
    h = h + jnp.where(sv > 0, s1, s0)
    mean = jnp.mean(h, axis=1, keepdims=True)
    var = jnp.mean(h * h, axis=1, keepdims=True) - mean * mean
    hn = (h - mean) * lax.rsqrt(var + _EPS)
    o_ref[...] = hn * g_ref[...] + b_ref[...]


_ln_call = pl.pallas_call(
    _ln_body,
    out_shape=jax.ShapeDtypeStruct((_TOK, _D), jnp.float32),
    grid=(_PB, _B),
    in_specs=[
        pl.BlockSpec((_TB, _D), lambda p, b: (b * _PB + p, 0)),
        pl.BlockSpec((_TB, _D), lambda p, b: (p, 0)),
        pl.BlockSpec((8, _D),
                     lambda p, b: (p * (_TB // 8) + _TB // 8, 0)),
        pl.BlockSpec((_TB, 1), lambda p, b: (b * _PB + p, 0)),
        pl.BlockSpec((2, _D), lambda p, b: (0, 0)),
        pl.BlockSpec((1, _D), lambda p, b: (0, 0)),
        pl.BlockSpec((1, _D), lambda p, b: (0, 0)),
    ],
    out_specs=pl.BlockSpec((_TB, _D), lambda p, b: (b * _PB + p, 0)),
)


def kernel(x, seg, word_table, pos_table, seg_table, gamma, beta):
    xf = x.reshape(-1).astype(jnp.int32)
    rows = _gather_rows(xf, word_table)
    out = _ln_call(rows, pos_table, pos_table,
                   seg.reshape(-1, 1).astype(jnp.int32),
                   seg_table, gamma.reshape(1, _D), beta.reshape(1, _D))
    return out.reshape(_B, _S, _D)


# speedup vs baseline: 2.5744x; 1.0271x over previous
import functools

import jax
import jax.numpy as jnp
from jax import lax
from jax.experimental import pallas as pl
from jax.experimental.pallas import tpu as pltpu
from jax.experimental.pallas import tpu_sc as plsc

_B, _S, _D = 4, 2048, 768
_PAD = 1
_EPS = 1e-12
_NC, _NS = 2, 16
_NW = _NC * _NS
_TOK = _B * _S
_TPW = _TOK // _NW
_K = 16
_NCH = _TPW // _K
_NBUF = 4
_TB = 2048

_mesh = plsc.VectorSubcoreMesh(core_axis_name="c", subcore_axis_name="s")


@functools.partial(
    pl.kernel,
    out_type=jax.ShapeDtypeStruct((_TOK, _D), jnp.float32),
    mesh=_mesh,
    scratch_types=[
        pltpu.VMEM((_TPW,), jnp.int32),
        pltpu.VMEM((_NBUF, _K, _D), jnp.float32),
        pltpu.SemaphoreType.DMA,
        pltpu.SemaphoreType.DMA,
        pltpu.SemaphoreType.DMA,
        pltpu.SemaphoreType.DMA,
        pltpu.SemaphoreType.DMA,
        pltpu.SemaphoreType.DMA,
        pltpu.SemaphoreType.DMA,
        pltpu.SemaphoreType.DMA,
    ],
)
def _gather_rows(x_hbm, wt_hbm, out_hbm, idx_v, rows_v,
                 g0, g1, g2, g3, o0, o1, o2, o3):
    wid = lax.axis_index("s") * _NC + lax.axis_index("c")
    base = wid * _TPW
    gsems = (g0, g1, g2, g3)
    osems = (o0, o1, o2, o3)

    pltpu.sync_copy(x_hbm.at[pl.ds(base, _TPW)], idx_v)

    def gather_desc(buf, tk0):
        return pltpu.make_async_copy(
            wt_hbm.at[idx_v.at[pl.ds(tk0, _K)]], rows_v.at[buf], gsems[buf])

    def out_desc(buf, tk0):
        return pltpu.make_async_copy(
            rows_v.at[buf], out_hbm.at[pl.ds(base + tk0, _K)], osems[buf])

    for b in range(_NBUF - 1):
        gather_desc(b, b * _K).start()

    def step(c4, carry0):
        for sub in range(_NBUF):
            c = c4 * _NBUF + sub
            tk0 = pl.multiple_of(c * _K, _K)
            gather_desc(sub, tk0).wait()
            out_desc(sub, tk0).start()

            @pl.when(c + _NBUF - 1 < _NCH)
            def _():
                @pl.when(c >= 1)
                def _():
                    prev = pl.multiple_of((c - 1) * _K, _K)
                    out_desc((sub + _NBUF - 1) % _NBUF, prev).wait()
                nxt = pl.multiple_of((c + _NBUF - 1) * _K, _K)
                gather_desc((sub + _NBUF - 1) % _NBUF, nxt).start()
        return carry0

    lax.fori_loop(0, _NCH // _NBUF, step, 0)

    for c in range(_NCH - _NBUF, _NCH):
        out_desc(c % _NBUF, c * _K).wait()


_PB = _S // _TB


def _ln_body(rows_ref, pos_ref, posn_ref, segid_ref, segtab_ref, g_ref,
             b_ref, o_ref):
    pos = jnp.concatenate(
        [pos_ref[_PAD + 1:, :], posn_ref[:_PAD + 1, :]], axis=0)
    h = rows_ref[...] + pos
    sv = segid_ref[...]
    s0 = segtab_ref[0:1, :]
    s1 = segtab_ref[1:2, :]
    h = h + jnp.where(sv > 0, s1, s0)
    mean = jnp.mean(h, axis=1, keepdims=True)
    var = jnp.mean(h * h, axis=1, keepdims=True) - mean * mean
    hn = (h - mean) * lax.rsqrt(var + _EPS)
    o_ref[...] = hn * g_ref[...] + b_ref[...]


_ln_call = pl.pallas_call(
    _ln_body,
    out_shape=jax.ShapeDtypeStruct((_TOK, _D), jnp.float32),
    grid=(_PB, _B),
    in_specs=[
        pl.BlockSpec((_TB, _D), lambda p, b: (b * _PB + p, 0)),
        pl.BlockSpec((_TB, _D), lambda p, b: (p, 0)),
        pl.BlockSpec((8, _D),
                     lambda p, b: (p * (_TB // 8) + _TB // 8, 0)),
        pl.BlockSpec((_TB, 1), lambda p, b: (b * _PB + p, 0)),
        pl.BlockSpec((2, _D), lambda p, b: (0, 0)),
        pl.BlockSpec((1, _D), lambda p, b: (0, 0)),
        pl.BlockSpec((1, _D), lambda p, b: (0, 0)),
    ],
    out_specs=pl.BlockSpec((_TB, _D), lambda p, b: (b * _PB + p, 0)),
)


def kernel(x, seg, word_table, pos_table, seg_table, gamma, beta):
    xf = x.reshape(-1).astype(jnp.int32)
    rows = _gather_rows(xf, word_table)
    out = _ln_call(rows, pos_table, pos_table,
                   seg.reshape(-1, 1).astype(jnp.int32),
                   seg_table, gamma.reshape(1, _D), beta.reshape(1, _D))
    return out.reshape(_B, _S, _D)


# speedup vs baseline: 2.5879x; 1.0052x over previous
import functools

import jax
import jax.numpy as jnp
from jax import lax
from jax.experimental import pallas as pl
from jax.experimental.pallas import tpu as pltpu
from jax.experimental.pallas import tpu_sc as plsc

_B, _S, _D = 4, 2048, 768
_PAD = 1
_EPS = 1e-12
_NC, _NS = 2, 16
_NW = _NC * _NS
_TOK = _B * _S
_TPW = _TOK // _NW
_K = 32
_NCH = _TPW // _K
_NBUF = 4
_TB = 2048

_mesh = plsc.VectorSubcoreMesh(core_axis_name="c", subcore_axis_name="s")


@functools.partial(
    pl.kernel,
    out_type=jax.ShapeDtypeStruct((_TOK, _D), jnp.float32),
    mesh=_mesh,
    scratch_types=[
        pltpu.VMEM((_TPW,), jnp.int32),
        pltpu.VMEM((_NBUF, _K, _D), jnp.float32),
        pltpu.SemaphoreType.DMA,
        pltpu.SemaphoreType.DMA,
        pltpu.SemaphoreType.DMA,
        pltpu.SemaphoreType.DMA,
        pltpu.SemaphoreType.DMA,
        pltpu.SemaphoreType.DMA,
        pltpu.SemaphoreType.DMA,
        pltpu.SemaphoreType.DMA,
    ],
)
def _gather_rows(x_hbm, wt_hbm, out_hbm, idx_v, rows_v,
                 g0, g1, g2, g3, o0, o1, o2, o3):
    wid = lax.axis_index("s") * _NC + lax.axis_index("c")
    base = wid * _TPW
    gsems = (g0, g1, g2, g3)
    osems = (o0, o1, o2, o3)

    pltpu.sync_copy(x_hbm.at[pl.ds(base, _TPW)], idx_v)

    def gather_desc(buf, tk0):
        return pltpu.make_async_copy(
            wt_hbm.at[idx_v.at[pl.ds(tk0, _K)]], rows_v.at[buf], gsems[buf])

    def out_desc(buf, tk0):
        return pltpu.make_async_copy(
            rows_v.at[buf], out_hbm.at[pl.ds(base + tk0, _K)], osems[buf])

    for b in range(_NBUF - 1):
        gather_desc(b, b * _K).start()

    def step(c4, carry0):
        for sub in range(_NBUF):
            c = c4 * _NBUF + sub
            tk0 = pl.multiple_of(c * _K, _K)
            gather_desc(sub, tk0).wait()
            out_desc(sub, tk0).start()

            @pl.when(c + _NBUF - 1 < _NCH)
            def _():
                @pl.when(c >= 1)
                def _():
                    prev = pl.multiple_of((c - 1) * _K, _K)
                    out_desc((sub + _NBUF - 1) % _NBUF, prev).wait()
                nxt = pl.multiple_of((c + _NBUF - 1) * _K, _K)
                gather_desc((sub + _NBUF - 1) % _NBUF, nxt).start()
        return carry0

    lax.fori_loop(0, _NCH // _NBUF, step, 0)

    for c in range(_NCH - _NBUF, _NCH):
        out_desc(c % _NBUF, c * _K).wait()


_PB = _S // _TB


def _ln_body(rows_ref, pos_ref, posn_ref, segid_ref, segtab_ref, g_ref,
             b_ref, o_ref):
    pos = jnp.concatenate(
        [pos_ref[_PAD + 1:, :], posn_ref[:_PAD + 1, :]], axis=0)
    h = rows_ref[...] + pos
    sv = segid_ref[...]
    s0 = segtab_ref[0:1, :]
    s1 = segtab_ref[1:2, :]
    h = h + jnp.where(sv > 0, s1, s0)
    mean = jnp.mean(h, axis=1, keepdims=True)
    var = jnp.mean(h * h, axis=1, keepdims=True) - mean * mean
    hn = (h - mean) * lax.rsqrt(var + _EPS)
    o_ref[...] = hn * g_ref[...] + b_ref[...]


_ln_call = pl.pallas_call(
    _ln_body,
    out_shape=jax.ShapeDtypeStruct((_TOK, _D), jnp.float32),
    grid=(_PB, _B),
    in_specs=[
        pl.BlockSpec((_TB, _D), lambda p, b: (b * _PB + p, 0)),
        pl.BlockSpec((_TB, _D), lambda p, b: (p, 0)),
        pl.BlockSpec((8, _D),
                     lambda p, b: (p * (_TB // 8) + _TB // 8, 0)),
        pl.BlockSpec((_TB, 1), lambda p, b: (b * _PB + p, 0)),
        pl.BlockSpec((2, _D), lambda p, b: (0, 0)),
        pl.BlockSpec((1, _D), lambda p, b: (0, 0)),
        pl.BlockSpec((1, _D), lambda p, b: (0, 0)),
    ],
    out_specs=pl.BlockSpec((_TB, _D), lambda p, b: (b * _PB + p, 0)),
)


def kernel(x, seg, word_table, pos_table, seg_table, gamma, beta):
    xf = x.reshape(-1).astype(jnp.int32)
    rows = _gather_rows(xf, word_table)
    out = _ln_call(rows, pos_table, pos_table,
                   seg.reshape(-1, 1).astype(jnp.int32),
                   seg_table, gamma.reshape(1, _D), beta.reshape(1, _D))
    return out.reshape(_B, _S, _D)


# speedup vs baseline: 2.5963x; 1.0033x over previous
import functools

import jax
import jax.numpy as jnp
from jax import lax
from jax.experimental import pallas as pl
from jax.experimental.pallas import tpu as pltpu
from jax.experimental.pallas import tpu_sc as plsc

_B, _S, _D = 4, 2048, 768
_PAD = 1
_EPS = 1e-12
_NC, _NS = 2, 16
_NW = _NC * _NS
_TOK = _B * _S
_TPW = _TOK // _NW
_K = 32
_NCH = _TPW // _K
_NBUF = 4
_TB = 2048

_mesh = plsc.VectorSubcoreMesh(core_axis_name="c", subcore_axis_name="s")


@functools.partial(
    pl.kernel,
    out_type=jax.ShapeDtypeStruct((_TOK, _D), jnp.float32),
    mesh=_mesh,
    scratch_types=[
        pltpu.VMEM((_TPW,), jnp.int32),
        pltpu.VMEM((_NBUF, _K, _D), jnp.float32),
        pltpu.SemaphoreType.DMA,
        pltpu.SemaphoreType.DMA,
        pltpu.SemaphoreType.DMA,
        pltpu.SemaphoreType.DMA,
        pltpu.SemaphoreType.DMA,
        pltpu.SemaphoreType.DMA,
        pltpu.SemaphoreType.DMA,
        pltpu.SemaphoreType.DMA,
    ],
)
def _gather_rows(x_hbm, wt_hbm, out_hbm, idx_v, rows_v,
                 g0, g1, g2, g3, o0, o1, o2, o3):
    wid = lax.axis_index("s") * _NC + lax.axis_index("c")
    base = wid * _TPW
    gsems = (g0, g1, g2, g3)
    osems = (o0, o1, o2, o3)

    pltpu.sync_copy(x_hbm.at[pl.ds(base, _TPW)], idx_v)

    def gather_desc(buf, tk0):
        return pltpu.make_async_copy(
            wt_hbm.at[idx_v.at[pl.ds(tk0, _K)]], rows_v.at[buf], gsems[buf])

    def out_desc(buf, tk0):
        return pltpu.make_async_copy(
            rows_v.at[buf], out_hbm.at[pl.ds(base + tk0, _K)], osems[buf])

    for b in range(_NBUF - 1):
        gather_desc(b, b * _K).start()

    def step(c4, carry0):
        for sub in range(_NBUF):
            c = c4 * _NBUF + sub
            tk0 = pl.multiple_of(c * _K, _K)
            gather_desc(sub, tk0).wait()
            out_desc(sub, tk0).start()

            @pl.when(c + _NBUF - 1 < _NCH)
            def _():
                @pl.when(c >= 1)
                def _():
                    prev = pl.multiple_of((c - 1) * _K, _K)
                    out_desc((sub + _NBUF - 1) % _NBUF, prev).wait()
                nxt = pl.multiple_of((c + _NBUF - 1) * _K, _K)
                gather_desc((sub + _NBUF - 1) % _NBUF, nxt).start()
        return carry0

    lax.fori_loop(0, _NCH // _NBUF, step, 0)

    for c in range(_NCH - _NBUF, _NCH):
        out_desc(c % _NBUF, c * _K).wait()


_PB = _S // _TB


def _ln_body(rows_ref, pos_ref, posn_ref, segid_ref, segtab_ref, g_ref,
             b_ref, o_ref, pshift_ref):
    @pl.when(pl.program_id(1) == 0)
    def _():
        pshift_ref[...] = jnp.concatenate(
            [pos_ref[_PAD + 1:, :], posn_ref[:_PAD + 1, :]], axis=0)

    h = rows_ref[...] + pshift_ref[...]
    sv = segid_ref[...]
    s0 = segtab_ref[0:1, :]
    s1 = segtab_ref[1:2, :]
    h = h + jnp.where(sv > 0, s1, s0)
    mean = jnp.mean(h, axis=1, keepdims=True)
    var = jnp.mean(h * h, axis=1, keepdims=True) - mean * mean
    hn = (h - mean) * lax.rsqrt(var + _EPS)
    o_ref[...] = hn * g_ref[...] + b_ref[...]


_ln_call = pl.pallas_call(
    _ln_body,
    out_shape=jax.ShapeDtypeStruct((_TOK, _D), jnp.float32),
    grid=(_PB, _B),
    in_specs=[
        pl.BlockSpec((_TB, _D), lambda p, b: (b * _PB + p, 0)),
        pl.BlockSpec((_TB, _D), lambda p, b: (p, 0)),
        pl.BlockSpec((8, _D),
                     lambda p, b: (p * (_TB // 8) + _TB // 8, 0)),
        pl.BlockSpec((_TB, 1), lambda p, b: (b * _PB + p, 0)),
        pl.BlockSpec((2, _D), lambda p, b: (0, 0)),
        pl.BlockSpec((1, _D), lambda p, b: (0, 0)),
        pl.BlockSpec((1, _D), lambda p, b: (0, 0)),
    ],
    out_specs=pl.BlockSpec((_TB, _D), lambda p, b: (b * _PB + p, 0)),
    scratch_shapes=[pltpu.VMEM((_TB, _D), jnp.float32)],
)


def kernel(x, seg, word_table, pos_table, seg_table, gamma, beta):
    xf = x.reshape(-1).astype(jnp.int32)
    rows = _gather_rows(xf, word_table)
    out = _ln_call(rows, pos_table, pos_table,
                   seg.reshape(-1, 1).astype(jnp.int32),
                   seg_table, gamma.reshape(1, _D), beta.reshape(1, _D))
    return out.reshape(_B, _S, _D)
